# baseline (speedup 1.0000x reference)
import functools

import jax
import jax.numpy as jnp
from jax import lax
from jax.experimental import pallas as pl
from jax.experimental.pallas import tpu as pltpu
from jax.experimental.pallas import tpu_sc as plsc

_BATCH = 16384
_EMB = 64
_BACKEND = 128
_NCLS = 128

_CH = 128


def _build_gather(batch, emb):
    info = plsc.get_sparse_core_info()
    nw = info.num_cores * info.num_subcores
    b_per_w = batch // nw
    n_ch = b_per_w // _CH
    mesh = plsc.VectorSubcoreMesh(core_axis_name="c", subcore_axis_name="s")

    @functools.partial(
        pl.kernel,
        mesh=mesh,
        compiler_params=pltpu.CompilerParams(use_tc_tiling_on_sc=False),
        out_type=jax.ShapeDtypeStruct((batch, emb), jnp.float32),
        scratch_types=[
            pltpu.VMEM((n_ch, _CH), jnp.int32),
            pltpu.VMEM((_CH, emb), jnp.float32),
            pltpu.SemaphoreType.DMA,
        ],
    )
    def gather(idx_hbm, table_hbm, out_hbm, idx_v, rows_v, sem):
        wid = lax.axis_index("s") * info.num_cores + lax.axis_index("c")
        base = wid * b_per_w
        pltpu.sync_copy(idx_hbm.at[pl.ds(wid * n_ch, n_ch)], idx_v)
        for j in range(n_ch):
            pltpu.async_copy(table_hbm.at[idx_v.at[j]], rows_v, sem).wait()
            pltpu.sync_copy(rows_v, out_hbm.at[pl.ds(base + j * _CH, _CH)])

    return gather, nw, n_ch


_gather, _NW, _NCH = _build_gather(_BATCH, _EMB)


def _mm_body(x_ref, f_ref, w1_ref, w2_ref, b_ref, o_ref):
    o_ref[...] = (
        jnp.dot(x_ref[...], w1_ref[...], preferred_element_type=jnp.float32)
        + jnp.dot(f_ref[...], w2_ref[...], preferred_element_type=jnp.float32)
        + b_ref[...]
    )


def kernel(batched_data, class_emb, W, b, domains):
    idx2d = domains.reshape(_NW * _NCH, _CH)
    feat = _gather(idx2d, class_emb)

    w1t = W[:, :_BACKEND].T
    w2t = W[:, _BACKEND:].T
    b2d = b.reshape(1, _NCLS)

    bm = 2048
    out = pl.pallas_call(
        _mm_body,
        grid=(_BATCH // bm,),
        in_specs=[
            pl.BlockSpec((bm, _BACKEND), lambda i: (i, 0)),
            pl.BlockSpec((bm, _EMB), lambda i: (i, 0)),
            pl.BlockSpec((_BACKEND, _NCLS), lambda i: (0, 0)),
            pl.BlockSpec((_EMB, _NCLS), lambda i: (0, 0)),
            pl.BlockSpec((1, _NCLS), lambda i: (0, 0)),
        ],
        out_specs=pl.BlockSpec((bm, _NCLS), lambda i: (i, 0)),
        out_shape=jax.ShapeDtypeStruct((_BATCH, _NCLS), jnp.float32),
    )(batched_data, feat, w1t, w2t, b2d)
    return out
